# baseline (speedup 1.0000x reference)
import functools

import jax
import jax.numpy as jnp
from jax import lax
from jax.experimental import pallas as pl
from jax.experimental.pallas import tpu as pltpu, tpu_sc as plsc

_R = 8
_K = 2 * _R + 1

_NC = 2
_NS = 16
_NW = _NC * _NS
_NPH = 8


@functools.lru_cache(maxsize=None)
def _build_fill(H: int, W: int):
    TH = 2 * H - 1
    TWP = ((2 * W - 1) + 15) // 16 * 16
    r0 = H - 1 - _R
    c0 = W - 1 - _R
    tiles = H * W
    per = tiles // _NW
    CH = 8
    nzc = TWP // 16

    mesh = plsc.VectorSubcoreMesh(
        core_axis_name="c", subcore_axis_name="s",
        num_cores=_NC, num_subcores=_NS)

    @functools.partial(
        pl.kernel,
        out_type=jax.ShapeDtypeStruct((1, 1, H, W, H, W), jnp.float32),
        mesh=mesh,
        scratch_types=[
            pltpu.VMEM((_K, _K), jnp.float32),
            pltpu.VMEM((TH, TWP), jnp.float32),
            pltpu.VMEM_SHARED((_NPH, TH, TWP), jnp.float32),
            pltpu.SemaphoreType.DMA,
        ],
        compiler_params=pltpu.CompilerParams(use_tc_tiling_on_sc=False),
    )
    def fill(biases_hbm, out_hbm, bv, tbuf, phases, sem):
        s = lax.axis_index("s")

        @pl.when(s < _NPH)
        def _build():
            pltpu.sync_copy(biases_hbm, bv)

            def zbody(r, carry):
                for j in range(nzc):
                    tbuf[r, pl.ds(j * 16, 16)] = jnp.zeros((16,), jnp.float32)
                return carry

            lax.fori_loop(0, TH, zbody, 0)
            for r in range(_K):
                tbuf[r0 + r, pl.ds(c0 - s, 16)] = bv[r, pl.ds(0, 16)]
                tbuf[r0 + r, pl.ds(c0 - s + 1, 16)] = bv[r, pl.ds(1, 16)]
            pltpu.sync_copy(tbuf, phases.at[s])

        plsc.subcore_barrier()

        wid = s * _NC + lax.axis_index("c")
        base = wid * per

        def obody(j, carry):
            p0 = base + j * CH
            cps = []
            for b in range(CH):
                p = p0 + b
                h1 = p // W
                w1 = p - h1 * W
                c = (W - 1) - w1
                ph = lax.rem(c, _NPH)
                cq = pl.multiple_of(c - ph, _NPH)
                cps.append(pltpu.async_copy(
                    phases.at[ph, pl.ds(H - 1 - h1, H), pl.ds(cq, W)],
                    out_hbm.at[0, 0, h1, w1],
                    sem))
            for cp in cps:
                cp.wait()
            return carry

        lax.fori_loop(0, per // CH, obody, 0)

    return fill


def kernel(feat, biases, all_h1s, all_w1s, all_h2s, all_w2s):
    H, W = feat.shape[-2], feat.shape[-1]
    out = _build_fill(H, W)(biases.astype(jnp.float32))
    return out.astype(feat.dtype)


# speedup vs baseline: 3.4080x; 3.4080x over previous
import functools

import jax
import jax.numpy as jnp
from jax import lax
from jax.experimental import pallas as pl
from jax.experimental.pallas import tpu as pltpu, tpu_sc as plsc

_R = 8
_K = 2 * _R + 1

_NC = 2
_NS = 16
_NW = _NC * _NS


@functools.lru_cache(maxsize=None)
def _build_fill(H: int, W: int):
    per = (H * W) // _NW
    ncs = per // H + 2
    nch = W // 16

    mesh = plsc.VectorSubcoreMesh(
        core_axis_name="c", subcore_axis_name="s",
        num_cores=_NC, num_subcores=_NS)

    @functools.partial(
        pl.kernel,
        out_type=jax.ShapeDtypeStruct((1, 1, H, W, H, W), jnp.float32),
        mesh=mesh,
        scratch_types=[
            pltpu.VMEM((_K, 32), jnp.float32),
            pltpu.VMEM((ncs * _K, W), jnp.float32),
            pltpu.VMEM((H, W), jnp.float32),
            pltpu.VMEM((H, W), jnp.float32),
            pltpu.SemaphoreType.DMA,
            pltpu.SemaphoreType.DMA,
        ],
    )
    def fill(biases_hbm, out_hbm, bv, band, tb0, tb1, sem0, sem1):
        pltpu.sync_copy(biases_hbm, bv)

        wid = lax.axis_index("s") * _NC + lax.axis_index("c")
        g0 = wid * per
        c0 = g0 // H

        zeros16 = jnp.zeros((16,), jnp.float32)

        def zrow(r, carry):
            for j in range(nch):
                tb0[r, pl.ds(j * 16, 16)] = zeros16
                tb1[r, pl.ds(j * 16, 16)] = zeros16
            return carry

        lax.fori_loop(0, H, zrow, 0)

        for ci in range(ncs):
            c = c0 + ci

            @pl.when(c < W)
            def _build(c=c, ci=ci):
                for kh in range(_K):
                    blo = bv[kh, pl.ds(0, 16)]
                    bhi = bv[kh, pl.ds(16, 16)]
                    for j in range(nch):
                        idx = lax.iota(jnp.int32, 16) + (j * 16 + c - (W - 1 - _R))
                        acc = jnp.zeros((16,), jnp.float32)
                        for kw in range(_K):
                            b = blo[kw] if kw < 16 else bhi[kw - 16]
                            acc = jnp.where(idx == kw, b, acc)
                        band[ci * _K + kh, pl.ds(j * 16, 16)] = acc

        def build_tile(g, tb):
            c = g // H
            h1 = g - c * H
            w1 = (W - 1) - c
            ci = c - c0

            @pl.when(h1 < 2)
            def _fullzero():
                def zr(r, carry):
                    for j in range(nch):
                        tb[r, pl.ds(j * 16, 16)] = zeros16
                    return carry
                lax.fori_loop(0, H, zr, 0)

            @pl.when(h1 >= 2)
            def _zstale():
                for d in (_R + 2, _R + 1):
                    @pl.when(h1 - d >= 0)
                    def _z(d=d):
                        for j in range(nch):
                            tb[h1 - d, pl.ds(j * 16, 16)] = zeros16

            for kh in range(_K):
                h2 = h1 - _R + kh

                @pl.when((h2 >= 0) & (h2 < H))
                def _row(kh=kh, h2=h2):
                    row = ci * _K + kh
                    for j in range(nch):
                        tb[h2, pl.ds(j * 16, 16)] = band[row, pl.ds(j * 16, 16)]

            return h1, w1

        def dst(h1, w1):
            return out_hbm.at[0, 0, h1, w1]

        h1a, w1a = build_tile(g0, tb0)
        pltpu.async_copy(tb0, dst(h1a, w1a), sem0)
        h1b, w1b = build_tile(g0 + 1, tb1)
        pltpu.async_copy(tb1, dst(h1b, w1b), sem1)

        def obody(t, carry):
            ga = g0 + 2 * t
            ca = ga // H
            pltpu.make_async_copy(tb0, dst(ga - ca * H, (W - 1) - ca), sem0).wait()
            h1, w1 = build_tile(ga, tb0)
            pltpu.async_copy(tb0, dst(h1, w1), sem0)

            gb = ga + 1
            cb = gb // H
            pltpu.make_async_copy(tb1, dst(gb - cb * H, (W - 1) - cb), sem1).wait()
            h1, w1 = build_tile(gb, tb1)
            pltpu.async_copy(tb1, dst(h1, w1), sem1)
            return carry

        lax.fori_loop(1, per // 2, obody, 0)

        gl = g0 + per - 2
        cl = gl // H
        pltpu.make_async_copy(tb0, dst(gl - cl * H, (W - 1) - cl), sem0).wait()
        gl = g0 + per - 1
        cl = gl // H
        pltpu.make_async_copy(tb1, dst(gl - cl * H, (W - 1) - cl), sem1).wait()

    return fill


def kernel(feat, biases, all_h1s, all_w1s, all_h2s, all_w2s):
    H, W = feat.shape[-2], feat.shape[-1]
    bp = jnp.pad(biases.astype(jnp.float32), ((0, 0), (0, 32 - _K)))
    out = _build_fill(H, W)(bp)
    return out.astype(feat.dtype)


# speedup vs baseline: 3.4336x; 1.0075x over previous
import functools

import jax
import jax.numpy as jnp
from jax import lax
from jax.experimental import pallas as pl
from jax.experimental.pallas import tpu as pltpu, tpu_sc as plsc

_R = 8
_K = 2 * _R + 1

_NC = 2
_NS = 16
_NW = _NC * _NS
_NB = 4


@functools.lru_cache(maxsize=None)
def _build_fill(H: int, W: int):
    per = (H * W) // _NW
    ncs = per // H + 2
    nch = W // 16

    mesh = plsc.VectorSubcoreMesh(
        core_axis_name="c", subcore_axis_name="s",
        num_cores=_NC, num_subcores=_NS)

    @functools.partial(
        pl.kernel,
        out_type=jax.ShapeDtypeStruct((1, 1, H, W, H, W), jnp.float32),
        mesh=mesh,
        scratch_types=[
            pltpu.VMEM((_K, 32), jnp.float32),
            pltpu.VMEM((ncs * _K, W), jnp.float32),
            [pltpu.VMEM((H, W), jnp.float32)] * _NB,
            [pltpu.SemaphoreType.DMA] * _NB,
        ],
    )
    def fill(biases_hbm, out_hbm, bv, band, tbs, sems):
        pltpu.sync_copy(biases_hbm, bv)

        wid = lax.axis_index("s") * _NC + lax.axis_index("c")
        g0 = wid * per
        c0 = g0 // H

        zeros16 = jnp.zeros((16,), jnp.float32)

        def zrow(r, carry):
            for tb in tbs:
                for j in range(nch):
                    tb[r, pl.ds(j * 16, 16)] = zeros16
            return carry

        lax.fori_loop(0, H, zrow, 0)

        for ci in range(ncs):
            c = c0 + ci

            @pl.when(c < W)
            def _build(c=c, ci=ci):
                for kh in range(_K):
                    blo = bv[kh, pl.ds(0, 16)]
                    bhi = bv[kh, pl.ds(16, 16)]
                    for j in range(nch):
                        idx = lax.iota(jnp.int32, 16) + (j * 16 + c - (W - 1 - _R))
                        acc = jnp.zeros((16,), jnp.float32)
                        for kw in range(_K):
                            b = blo[kw] if kw < 16 else bhi[kw - 16]
                            acc = jnp.where(idx == kw, b, acc)
                        band[ci * _K + kh, pl.ds(j * 16, 16)] = acc

        def build_tile(g, tb):
            c = g // H
            h1 = g - c * H
            w1 = (W - 1) - c
            ci = c - c0

            @pl.when(h1 < _NB)
            def _fullzero():
                def zr(r, carry):
                    for j in range(nch):
                        tb[r, pl.ds(j * 16, 16)] = zeros16
                    return carry
                lax.fori_loop(0, H, zr, 0)

            @pl.when(h1 >= _NB)
            def _zstale():
                for d in range(_R + _NB, _R, -1):
                    @pl.when(h1 - d >= 0)
                    def _z(d=d):
                        for j in range(nch):
                            tb[h1 - d, pl.ds(j * 16, 16)] = zeros16

            def krow(kh, carry):
                h2 = h1 - _R + kh

                @pl.when((h2 >= 0) & (h2 < H))
                def _row():
                    row = ci * _K + kh
                    for j in range(nch):
                        tb[h2, pl.ds(j * 16, 16)] = band[row, pl.ds(j * 16, 16)]

                return carry

            lax.fori_loop(0, _K, krow, 0)
            return h1, w1

        def dst(h1, w1):
            return out_hbm.at[0, 0, h1, w1]

        for b in range(_NB):
            h1a, w1a = build_tile(g0 + b, tbs[b])
            pltpu.async_copy(tbs[b], dst(h1a, w1a), sems[b])

        def obody(t, carry):
            for b in range(_NB):
                g = g0 + _NB * t + b
                cg = g // H
                pltpu.make_async_copy(
                    tbs[b], dst(g - cg * H, (W - 1) - cg), sems[b]).wait()
                h1, w1 = build_tile(g, tbs[b])
                pltpu.async_copy(tbs[b], dst(h1, w1), sems[b])
            return carry

        lax.fori_loop(1, per // _NB, obody, 0)

        for b in range(_NB):
            gl = g0 + per - _NB + b
            cl = gl // H
            pltpu.make_async_copy(
                tbs[b], dst(gl - cl * H, (W - 1) - cl), sems[b]).wait()

    return fill


def kernel(feat, biases, all_h1s, all_w1s, all_h2s, all_w2s):
    H, W = feat.shape[-2], feat.shape[-1]
    bp = jnp.pad(biases.astype(jnp.float32), ((0, 0), (0, 32 - _K)))
    out = _build_fill(H, W)(bp)
    return out.astype(feat.dtype)


# speedup vs baseline: 3.4851x; 1.0150x over previous
import functools

import jax
import jax.numpy as jnp
from jax import lax
from jax.experimental import pallas as pl
from jax.experimental.pallas import tpu as pltpu, tpu_sc as plsc

_R = 8
_K = 2 * _R + 1

_NC = 2
_NS = 16
_NW = _NC * _NS
_NB = 4


@functools.lru_cache(maxsize=None)
def _build_fill(H: int, W: int):
    per = (H * W) // _NW
    ncs = per // H + 2
    nch = W // 16

    mesh = plsc.VectorSubcoreMesh(
        core_axis_name="c", subcore_axis_name="s",
        num_cores=_NC, num_subcores=_NS)

    @functools.partial(
        pl.kernel,
        out_type=jax.ShapeDtypeStruct((1, 1, H, W, H, W), jnp.float32),
        mesh=mesh,
        scratch_types=[
            pltpu.VMEM((_K, _K), jnp.float32),
            pltpu.VMEM((ncs * _K, W), jnp.float32),
            [pltpu.VMEM((H, W), jnp.float32)] * _NB,
            [pltpu.SemaphoreType.DMA] * _NB,
        ],
    )
    def fill(biases_hbm, out_hbm, bv, band, tbs, sems):
        pltpu.sync_copy(biases_hbm, bv)

        wid = lax.axis_index("s") * _NC + lax.axis_index("c")
        g0 = wid * per
        c0 = g0 // H

        zeros16 = jnp.zeros((16,), jnp.float32)

        def zrow(r, carry):
            for tb in tbs:
                for j in range(nch):
                    tb[r, pl.ds(j * 16, 16)] = zeros16
            return carry

        lax.fori_loop(0, H, zrow, 0)

        for ci in range(ncs):
            c = c0 + ci

            @pl.when(c < W)
            def _build(c=c, ci=ci):
                cc = c - (W - 1 - _R)
                for kh in range(_K):
                    blo = bv[kh, pl.ds(0, 16)]
                    bhi = bv[kh, pl.ds(1, 16)]
                    for j in range(nch):
                        lo = 16 * j + cc
                        hit = (lo <= _K - 1) & (lo >= -15)

                        @pl.when(hit)
                        def _sel(j=j, lo=lo):
                            idx = lax.iota(jnp.int32, 16) + lo
                            acc = jnp.zeros((16,), jnp.float32)
                            for kw in range(_K):
                                b = blo[kw] if kw < 16 else bhi[15]
                                acc = jnp.where(idx == kw, b, acc)
                            band[ci * _K + kh, pl.ds(j * 16, 16)] = acc

                        @pl.when(jnp.logical_not(hit))
                        def _zero(j=j):
                            band[ci * _K + kh, pl.ds(j * 16, 16)] = zeros16

        def build_tile(g, tb):
            c = g // H
            h1 = g - c * H
            w1 = (W - 1) - c
            ci = c - c0

            @pl.when(h1 < _NB)
            def _fullzero():
                def zr(r, carry):
                    for j in range(nch):
                        tb[r, pl.ds(j * 16, 16)] = zeros16
                    return carry
                lax.fori_loop(0, H, zr, 0)

            @pl.when(h1 >= _NB)
            def _zstale():
                for d in range(_R + _NB, _R, -1):
                    @pl.when(h1 - d >= 0)
                    def _z(d=d):
                        for j in range(nch):
                            tb[h1 - d, pl.ds(j * 16, 16)] = zeros16

            def krow(kh, carry):
                h2 = h1 - _R + kh

                @pl.when((h2 >= 0) & (h2 < H))
                def _row():
                    row = ci * _K + kh
                    for j in range(nch):
                        tb[h2, pl.ds(j * 16, 16)] = band[row, pl.ds(j * 16, 16)]

                return carry

            lax.fori_loop(0, _K, krow, 0)
            return h1, w1

        def dst(h1, w1):
            return out_hbm.at[0, 0, h1, w1]

        for b in range(_NB):
            h1a, w1a = build_tile(g0 + b, tbs[b])
            pltpu.async_copy(tbs[b], dst(h1a, w1a), sems[b])

        def obody(t, carry):
            for b in range(_NB):
                g = g0 + _NB * t + b
                cg = g // H
                pltpu.make_async_copy(
                    tbs[b], dst(g - cg * H, (W - 1) - cg), sems[b]).wait()
                h1, w1 = build_tile(g, tbs[b])
                pltpu.async_copy(tbs[b], dst(h1, w1), sems[b])
            return carry

        lax.fori_loop(1, per // _NB, obody, 0)

        for b in range(_NB):
            gl = g0 + per - _NB + b
            cl = gl // H
            pltpu.make_async_copy(
                tbs[b], dst(gl - cl * H, (W - 1) - cl), sems[b]).wait()

    return fill


def kernel(feat, biases, all_h1s, all_w1s, all_h2s, all_w2s):
    H, W = feat.shape[-2], feat.shape[-1]
    out = _build_fill(H, W)(biases.astype(jnp.float32))
    return out.astype(feat.dtype)


# speedup vs baseline: 3.7475x; 1.0753x over previous
import functools

import jax
import jax.numpy as jnp
from jax import lax
from jax.experimental import pallas as pl
from jax.experimental.pallas import tpu as pltpu, tpu_sc as plsc

_R = 8
_K = 2 * _R + 1

_NC = 2
_NS = 16
_NW = _NC * _NS
_BT = 4


@functools.lru_cache(maxsize=None)
def _build_fill(H: int, W: int):
    NBW = 4
    NBH = _NW // NBW
    BH = H // NBH
    BW = W // NBW
    nbt = BW // _BT
    nch = W // 16
    MW = (BW - 1 + W + 15) // 16 * 16

    mesh = plsc.VectorSubcoreMesh(
        core_axis_name="c", subcore_axis_name="s",
        num_cores=_NC, num_subcores=_NS)

    @functools.partial(
        pl.kernel,
        out_type=jax.ShapeDtypeStruct((1, 1, H, W, H, W), jnp.float32),
        mesh=mesh,
        scratch_types=[
            pltpu.VMEM((_K, _K), jnp.float32),
            pltpu.VMEM((_K, MW), jnp.float32),
            [pltpu.VMEM((_BT, H, W), jnp.float32)] * 2,
            [pltpu.SemaphoreType.DMA] * 2,
        ],
    )
    def fill(biases_hbm, out_hbm, bv, master, tbs, sems):
        pltpu.sync_copy(biases_hbm, bv)

        wid = lax.axis_index("s") * _NC + lax.axis_index("c")
        bh = wid // NBW
        bw = wid - bh * NBW
        h1base = bh * BH
        w1base = bw * BW
        cmin = (W - 1) - (w1base + BW - 1)

        zeros16 = jnp.zeros((16,), jnp.float32)

        def zrow(r, carry):
            for tb in tbs:
                for i in range(_BT):
                    for j in range(nch):
                        tb[i, r, pl.ds(j * 16, 16)] = zeros16
            return carry

        lax.fori_loop(0, H, zrow, 0)

        cc = cmin - (W - 1 - _R)
        for kh in range(_K):
            blo = bv[kh, pl.ds(0, 16)]
            bhi = bv[kh, pl.ds(1, 16)]
            for j in range(MW // 16):
                lo = 16 * j + cc
                hit = (lo <= _K - 1) & (lo >= -15)

                @pl.when(hit)
                def _sel(j=j, lo=lo, blo=blo, bhi=bhi, kh=kh):
                    idx = lax.iota(jnp.int32, 16) + lo
                    acc = jnp.zeros((16,), jnp.float32)
                    for kw in range(_K):
                        b = blo[kw] if kw < 16 else bhi[15]
                        acc = jnp.where(idx == kw, b, acc)
                    master[kh, pl.ds(j * 16, 16)] = acc

                @pl.when(jnp.logical_not(hit))
                def _zero(j=j, kh=kh):
                    master[kh, pl.ds(j * 16, 16)] = zeros16

        def build_batch(q, tb):
            qh = q // nbt
            h1 = h1base + qh
            bi = q - qh * nbt
            w1lo = w1base + _BT * bi

            @pl.when(h1 - (_R + 1) >= 0)
            def _zstale():
                for i in range(_BT):
                    for j in range(nch):
                        tb[i, h1 - (_R + 1), pl.ds(j * 16, 16)] = zeros16

            def krow(kh, carry):
                h2 = h1 - _R + kh

                @pl.when((h2 >= 0) & (h2 < H))
                def _row():
                    for i in range(_BT):
                        woff = (W - 1) - (w1lo + i) - cmin
                        for j in range(nch):
                            tb[i, h2, pl.ds(j * 16, 16)] = (
                                master[kh, pl.ds(woff + j * 16, 16)])

                return carry

            lax.fori_loop(0, _K, krow, 0)
            return h1, w1lo

        def dst(h1, w1lo):
            return out_hbm.at[0, 0, h1, pl.ds(w1lo, _BT)]

        nq = nbt * BH

        for b in range(2):
            h1a, w1a = build_batch(jnp.int32(b), tbs[b])
            pltpu.async_copy(tbs[b], dst(h1a, w1a), sems[b])

        def obody(t, carry):
            for b in range(2):
                q = 2 * t + b
                qh = q // nbt
                pltpu.make_async_copy(
                    tbs[b],
                    dst(h1base + qh, w1base + _BT * (q - qh * nbt)),
                    sems[b]).wait()
                h1, w1lo = build_batch(q, tbs[b])
                pltpu.async_copy(tbs[b], dst(h1, w1lo), sems[b])
            return carry

        lax.fori_loop(1, nq // 2, obody, 0)

        for b in range(2):
            q = nq - 2 + b
            qh = q // nbt
            pltpu.make_async_copy(
                tbs[b],
                dst(h1base + qh, w1base + _BT * (q - qh * nbt)),
                sems[b]).wait()

    return fill


def kernel(feat, biases, all_h1s, all_w1s, all_h2s, all_w2s):
    H, W = feat.shape[-2], feat.shape[-1]
    out = _build_fill(H, W)(biases.astype(jnp.float32))
    return out.astype(feat.dtype)
